# Initial kernel scaffold; baseline (speedup 1.0000x reference)
import jax
import jax.numpy as jnp
from jax.experimental import pallas as pl


def kernel(input_ids, token_type_ids, word_embeddings, position_embeddings, token_type_embeddings, sentence_type_embeddings, gamma, beta):
    raise NotImplementedError("write your pallas kernel here")


# speedup vs baseline: 12.3142x; 12.3142x over previous
import functools

import jax
import jax.numpy as jnp
from jax import lax
from jax.experimental import pallas as pl
from jax.experimental.pallas import tpu as pltpu
from jax.experimental.pallas import tpu_sc as plsc

_EPS = 1e-12
_NC = 2
_NS = 16
_NW = _NC * _NS


def _sc_gather(idx_flat, table, chunk=128):
    n = idx_flat.shape[0]
    h = table.shape[1]
    per_w = n // _NW
    n_chunks = per_w // chunk
    mesh = plsc.VectorSubcoreMesh(core_axis_name="c", subcore_axis_name="s")

    @functools.partial(
        pl.kernel,
        mesh=mesh,
        out_type=jax.ShapeDtypeStruct((n, h), table.dtype),
        scratch_types=[
            pltpu.VMEM((chunk,), jnp.int32),
            pltpu.VMEM((chunk, h), table.dtype),
            pltpu.SemaphoreType.DMA,
        ],
    )
    def gather_kernel(idx_hbm, table_hbm, out_hbm, idx_v, rows_v, sem):
        wid = lax.axis_index("s") * _NC + lax.axis_index("c")
        base = wid * per_w

        @pl.loop(0, n_chunks)
        def _(i):
            off = base + i * chunk
            pltpu.sync_copy(idx_hbm.at[pl.ds(off, chunk)], idx_v)
            pltpu.async_copy(table_hbm.at[idx_v], rows_v, sem).wait()
            pltpu.sync_copy(rows_v, out_hbm.at[pl.ds(off, chunk)])

    return gather_kernel(idx_flat, table)


def _tc_body(tt_ref, gath_ref, pos_ref, comb_ref, gamma_ref, beta_ref, out_ref):
    bb, l, h = gath_ref.shape
    nt = comb_ref.shape[0]
    tt = tt_ref[...]
    onehot = (
        tt[:, :, None] == lax.broadcasted_iota(jnp.int32, (1, 1, nt), 2)
    ).astype(jnp.float32)
    extra = lax.dot_general(
        onehot.reshape(bb * l, nt),
        comb_ref[...],
        dimension_numbers=(((1,), (0,)), ((), ())),
        preferred_element_type=jnp.float32,
    ).reshape(bb, l, h)
    emb = gath_ref[...] + pos_ref[...][None, :, :] + extra
    mu = jnp.mean(emb, axis=-1, keepdims=True)
    var = jnp.mean((emb - mu) ** 2, axis=-1, keepdims=True)
    norm = (emb - mu) * lax.rsqrt(var + _EPS)
    out_ref[...] = norm * gamma_ref[...][None, :, :] + beta_ref[...][None, :, :]


def _tc_finish(token_type_ids, gathered, pos, comb, gamma, beta, bb=16,
               interpret=False):
    b, l = token_type_ids.shape
    h = gathered.shape[-1]
    nt = comb.shape[0]
    grid = (b // bb,)
    return pl.pallas_call(
        _tc_body,
        grid=grid,
        in_specs=[
            pl.BlockSpec((bb, l), lambda i: (i, 0)),
            pl.BlockSpec((bb, l, h), lambda i: (i, 0, 0)),
            pl.BlockSpec((l, h), lambda i: (0, 0)),
            pl.BlockSpec((nt, h), lambda i: (0, 0)),
            pl.BlockSpec((1, h), lambda i: (0, 0)),
            pl.BlockSpec((1, h), lambda i: (0, 0)),
        ],
        out_specs=pl.BlockSpec((bb, l, h), lambda i: (i, 0, 0)),
        out_shape=jax.ShapeDtypeStruct((b, l, h), jnp.float32),
        interpret=interpret,
    )(token_type_ids, gathered, pos, comb, gamma, beta)


def kernel(input_ids, token_type_ids, word_embeddings, position_embeddings,
           token_type_embeddings, sentence_type_embeddings, gamma, beta):
    b, l = input_ids.shape
    h = word_embeddings.shape[1]
    n = b * l
    ids_flat = input_ids.astype(jnp.int32).reshape(n)
    tt = token_type_ids.astype(jnp.int32)

    ns = sentence_type_embeddings.shape[0]
    tok_rows = jnp.take(
        token_type_embeddings,
        (jnp.arange(ns) > 0).astype(jnp.int32), axis=0)
    comb = sentence_type_embeddings + tok_rows
    comb = jnp.concatenate(
        [comb, jnp.zeros((32 - ns, h), jnp.float32)], axis=0)

    pos = position_embeddings[:l]

    gathered = _sc_gather(ids_flat, word_embeddings).reshape(b, l, h)
    return _tc_finish(tt, gathered, pos, comb,
                      gamma.reshape(1, h), beta.reshape(1, h))
